# baseline (speedup 1.0000x reference)
import functools

import jax
import jax.numpy as jnp
from jax import lax
from jax.experimental import pallas as pl
from jax.experimental.pallas import tpu as pltpu
from jax.experimental.pallas import tpu_sc as plsc

_NC = 2
_NS = 16
_NW = _NC * _NS
_CH = 96
_NBUF = 2


def _row_pad(n):
    rows_per_tile = -(-n // (8 * _NS)) * 8
    return rows_per_tile, rows_per_tile * _NS


def _sc_bincount2(sidx3, didx3, n):
    nw, iters, ch = sidx3.shape
    assert nw == _NW and ch == _CH
    rows_per_tile, n_pad = _row_pad(n)

    mesh = plsc.VectorSubcoreMesh(core_axis_name="c", subcore_axis_name="s")

    @functools.partial(
        pl.kernel,
        mesh=mesh,
        compiler_params=pltpu.CompilerParams(use_tc_tiling_on_sc=False),
        out_type=[jax.ShapeDtypeStruct((2, n_pad, 16), jnp.float32),
                  jax.ShapeDtypeStruct((2, n_pad, 16), jnp.float32)],
        scratch_types=[
            pltpu.VMEM((iters, _CH), jnp.int32),
            pltpu.VMEM((iters, _CH), jnp.int32),
            pltpu.VMEM((_CH, 16), jnp.float32),
            pltpu.VMEM_SHARED((n_pad, 16), jnp.float32),
            pltpu.VMEM_SHARED((n_pad, 16), jnp.float32),
            pltpu.SemaphoreType.DMA,
            pltpu.SemaphoreType.DMA,
        ],
    )
    def k(sidx_hbm, didx_hbm, zeros_hbm, outs_hbm, outd_hbm, sidx, didx, ones,
          acc_s, acc_d, sem_s, sem_d):
        c = lax.axis_index("c")
        s = lax.axis_index("s")
        wid = c * _NS + s
        base_rows = s * rows_per_tile

        def fill_ones(i, carry):
            ones[i] = jnp.ones((16,), jnp.float32)
            return carry

        lax.fori_loop(0, _CH, fill_ones, 0)

        pltpu.sync_copy(zeros_hbm, acc_s.at[pl.ds(base_rows, rows_per_tile)])
        pltpu.sync_copy(zeros_hbm, acc_d.at[pl.ds(base_rows, rows_per_tile)])
        pltpu.sync_copy(sidx_hbm.at[wid], sidx)
        pltpu.sync_copy(didx_hbm.at[wid], didx)
        plsc.subcore_barrier()

        def step(i, carry):
            pltpu.async_copy(ones, acc_s.at[sidx.at[i]], sem_s, add=True)
            pltpu.async_copy(ones, acc_d.at[didx.at[i]], sem_d, add=True)

            @pl.when(i > 0)
            def _():
                pltpu.make_async_copy(ones, acc_s.at[sidx.at[i]], sem_s).wait()
                pltpu.make_async_copy(ones, acc_d.at[didx.at[i]], sem_d).wait()

            return carry

        lax.fori_loop(0, iters, step, 0)
        pltpu.make_async_copy(ones, acc_s.at[sidx.at[0]], sem_s).wait()
        pltpu.make_async_copy(ones, acc_d.at[didx.at[0]], sem_d).wait()

        plsc.subcore_barrier()
        rt = pl.ds(base_rows, rows_per_tile)
        pltpu.sync_copy(acc_s.at[rt], outs_hbm.at[c, rt])
        pltpu.sync_copy(acc_d.at[rt], outd_hbm.at[c, rt])

    return k(sidx3, didx3, jnp.zeros((rows_per_tile, 16), jnp.float32))


def _sc_scatter(m, sidx3, didx3):
    n, f = m.shape
    nw, iters, ch = sidx3.shape
    assert nw == _NW and ch == _CH and iters % _NBUF == 0
    rows_per_tile, n_pad = _row_pad(n)

    mesh = plsc.VectorSubcoreMesh(core_axis_name="c", subcore_axis_name="s")

    @functools.partial(
        pl.kernel,
        mesh=mesh,
        compiler_params=pltpu.CompilerParams(use_tc_tiling_on_sc=False),
        out_type=jax.ShapeDtypeStruct((2, n_pad, f), jnp.float32),
        scratch_types=[
            pltpu.VMEM((iters, _CH), jnp.int32),
            pltpu.VMEM((iters, _CH), jnp.int32),
        ] + [pltpu.VMEM((_CH, f), jnp.float32)] * _NBUF + [
            pltpu.VMEM_SHARED((n_pad, f), jnp.float32),
        ] + [pltpu.SemaphoreType.DMA] * (2 * _NBUF),
    )
    def k(m_hbm, src_hbm, dst_hbm, zeros_hbm, out_hbm, sidx, didx, *rest):
        rows = rest[:_NBUF]
        acc = rest[_NBUF]
        semg = rest[_NBUF + 1:2 * _NBUF + 1]
        sems = rest[2 * _NBUF + 1:]
        c = lax.axis_index("c")
        s = lax.axis_index("s")
        wid = c * _NS + s
        base_rows = s * rows_per_tile

        pltpu.sync_copy(zeros_hbm, acc.at[pl.ds(base_rows, rows_per_tile)])
        pltpu.sync_copy(src_hbm.at[wid], sidx)
        pltpu.sync_copy(dst_hbm.at[wid], didx)
        plsc.subcore_barrier()

        def gather_start(i, b):
            pltpu.async_copy(m_hbm.at[sidx.at[i]], rows[b], semg[b])

        def gather_wait(i, b):
            pltpu.make_async_copy(m_hbm.at[sidx.at[i]], rows[b], semg[b]).wait()

        def scatter_start(i, b):
            pltpu.async_copy(rows[b], acc.at[didx.at[i]], sems[b], add=True)

        def scatter_wait(i, b):
            pltpu.make_async_copy(rows[b], acc.at[didx.at[i]], sems[b]).wait()

        for b in range(_NBUF):
            gather_start(b, b)

        def grp(j, carry):
            i0 = j * _NBUF
            for b in range(_NBUF):
                gather_wait(i0 + b, b)
                scatter_start(i0 + b, b)
            for b in range(_NBUF):
                @pl.when(i0 + b + _NBUF < iters)
                def _(b=b):
                    scatter_wait(i0 + b, b)
                    gather_start(i0 + b + _NBUF, b)
            return carry

        lax.fori_loop(0, iters // _NBUF, grp, 0)
        for b in range(_NBUF):
            scatter_wait(iters - _NBUF + b, b)

        plsc.subcore_barrier()
        rt = pl.ds(base_rows, rows_per_tile)
        pltpu.sync_copy(acc.at[rt], out_hbm.at[c, rt])

    return k(m, sidx3, didx3, jnp.zeros((rows_per_tile, f), jnp.float32))


def _onorm(degs_ref):
    od = degs_ref[0, :, 0:1] + degs_ref[1, :, 0:1]
    return lax.rsqrt(jnp.maximum(od, 1.0))


def _tc_a_body(x_ref, w_ref, degs_ref, m1_ref):
    m1_ref[...] = jnp.dot(
        x_ref[...] * _onorm(degs_ref), w_ref[...],
        preferred_element_type=jnp.float32)


def _tc_b_body(p1_ref, odeg_ref, ideg_ref, b1_ref, w2_ref, m2_ref):
    inorm = _onorm(ideg_ref)
    h = (p1_ref[0] + p1_ref[1]) * inorm + b1_ref[...]
    h = jnp.maximum(h, 0.0)
    m2_ref[...] = jnp.dot(
        h, w2_ref[...], preferred_element_type=jnp.float32) * _onorm(odeg_ref)


def _tc_c_body(p2_ref, ideg_ref, b2_ref, out_ref):
    z = (p2_ref[0] + p2_ref[1]) * _onorm(ideg_ref) + b2_ref[...]
    zmax = jnp.max(z, axis=-1, keepdims=True)
    zs = z - zmax
    out_ref[...] = zs - jnp.log(jnp.sum(jnp.exp(zs), axis=-1, keepdims=True))


def kernel(x, edge_index, W1, b1, W2, b2):
    n, nfeat = x.shape
    nhid = W1.shape[1]
    ncls = W2.shape[1]
    bn = 2000
    nblk = n // bn
    assert nblk * bn == n

    src = edge_index[0].astype(jnp.int32)
    dst = edge_index[1].astype(jnp.int32)
    e = src.shape[0]
    _, n_pad = _row_pad(n)

    iters = -(-e // (_NW * _CH * _NBUF)) * _NBUF
    e_pad = _NW * iters * _CH
    pk = jnp.arange(e_pad - e, dtype=jnp.int32)
    trash = n + pk % (n_pad - n)
    src_gat3 = jnp.concatenate([src, pk % n]).reshape(_NW, iters, _CH)
    src_cnt3 = jnp.concatenate([src, trash]).reshape(_NW, iters, _CH)
    dst3 = jnp.concatenate([dst, trash]).reshape(_NW, iters, _CH)

    odegs, idegs = _sc_bincount2(src_cnt3, dst3, n)

    odeg_spec = pl.BlockSpec((2, bn, 16), lambda i: (0, i, 0))
    ideg_spec = pl.BlockSpec((2, bn, 16), lambda i: (0, i, 0))

    m1 = pl.pallas_call(
        _tc_a_body,
        grid=(nblk,),
        in_specs=[
            pl.BlockSpec((bn, nfeat), lambda i: (i, 0)),
            pl.BlockSpec((nfeat, nhid), lambda i: (0, 0)),
            odeg_spec,
        ],
        out_specs=pl.BlockSpec((bn, nhid), lambda i: (i, 0)),
        out_shape=jax.ShapeDtypeStruct((n, nhid), jnp.float32),
    )(x, W1, odegs)

    part1 = _sc_scatter(m1, src_gat3, dst3)

    m2 = pl.pallas_call(
        _tc_b_body,
        grid=(nblk,),
        in_specs=[
            pl.BlockSpec((2, bn, nhid), lambda i: (0, i, 0)),
            odeg_spec,
            ideg_spec,
            pl.BlockSpec((1, nhid), lambda i: (0, 0)),
            pl.BlockSpec((nhid, ncls), lambda i: (0, 0)),
        ],
        out_specs=pl.BlockSpec((bn, ncls), lambda i: (i, 0)),
        out_shape=jax.ShapeDtypeStruct((n, ncls), jnp.float32),
    )(part1, odegs, idegs, b1.reshape(1, nhid), W2)

    part2 = _sc_scatter(m2, src_gat3, dst3)

    out = pl.pallas_call(
        _tc_c_body,
        grid=(nblk,),
        in_specs=[
            pl.BlockSpec((2, bn, ncls), lambda i: (0, i, 0)),
            ideg_spec,
            pl.BlockSpec((1, ncls), lambda i: (0, 0)),
        ],
        out_specs=pl.BlockSpec((bn, ncls), lambda i: (i, 0)),
        out_shape=jax.ShapeDtypeStruct((n, ncls), jnp.float32),
    )(part2, idegs, b2.reshape(1, ncls))

    return out


# speedup vs baseline: 1.1721x; 1.1721x over previous
import functools

import jax
import jax.numpy as jnp
from jax import lax
from jax.experimental import pallas as pl
from jax.experimental.pallas import tpu as pltpu
from jax.experimental.pallas import tpu_sc as plsc

_NC = 2
_NS = 16
_NW = _NC * _NS
_CH = 96
_NBUF = 2


def _row_pad(n):
    rows_per_tile = -(-n // (8 * _NS)) * 8
    return rows_per_tile, rows_per_tile * _NS


def _sc_bincount2(sidx3, didx3, n):
    nw, iters, ch = sidx3.shape
    assert nw == _NW and ch == _CH
    rows_per_tile, n_pad = _row_pad(n)

    mesh = plsc.VectorSubcoreMesh(core_axis_name="c", subcore_axis_name="s")

    @functools.partial(
        pl.kernel,
        mesh=mesh,
        compiler_params=pltpu.CompilerParams(use_tc_tiling_on_sc=False),
        out_type=[jax.ShapeDtypeStruct((2, n_pad, 16), jnp.float32),
                  jax.ShapeDtypeStruct((2, n_pad, 16), jnp.float32)],
        scratch_types=[
            pltpu.VMEM((iters, _CH), jnp.int32),
            pltpu.VMEM((iters, _CH), jnp.int32),
            pltpu.VMEM((_CH, 16), jnp.float32),
            pltpu.VMEM_SHARED((n_pad, 16), jnp.float32),
            pltpu.VMEM_SHARED((n_pad, 16), jnp.float32),
            pltpu.SemaphoreType.DMA,
            pltpu.SemaphoreType.DMA,
        ],
    )
    def k(sidx_hbm, didx_hbm, zeros_hbm, outs_hbm, outd_hbm, sidx, didx, ones,
          acc_s, acc_d, sem_s, sem_d):
        c = lax.axis_index("c")
        s = lax.axis_index("s")
        wid = c * _NS + s
        base_rows = s * rows_per_tile

        def fill_ones(i, carry):
            ones[i] = jnp.ones((16,), jnp.float32)
            return carry

        lax.fori_loop(0, _CH, fill_ones, 0)

        pltpu.sync_copy(zeros_hbm, acc_s.at[pl.ds(base_rows, rows_per_tile)])
        pltpu.sync_copy(zeros_hbm, acc_d.at[pl.ds(base_rows, rows_per_tile)])
        pltpu.sync_copy(sidx_hbm.at[wid], sidx)
        pltpu.sync_copy(didx_hbm.at[wid], didx)
        plsc.subcore_barrier()

        def step(i, carry):
            pltpu.async_copy(ones, acc_s.at[sidx.at[i]], sem_s, add=True)
            pltpu.async_copy(ones, acc_d.at[didx.at[i]], sem_d, add=True)

            @pl.when(i > 0)
            def _():
                pltpu.make_async_copy(ones, acc_s.at[sidx.at[i]], sem_s).wait()
                pltpu.make_async_copy(ones, acc_d.at[didx.at[i]], sem_d).wait()

            return carry

        lax.fori_loop(0, iters, step, 0)
        pltpu.make_async_copy(ones, acc_s.at[sidx.at[0]], sem_s).wait()
        pltpu.make_async_copy(ones, acc_d.at[didx.at[0]], sem_d).wait()

        plsc.subcore_barrier()
        rt = pl.ds(base_rows, rows_per_tile)
        pltpu.sync_copy(acc_s.at[rt], outs_hbm.at[c, rt])
        pltpu.sync_copy(acc_d.at[rt], outd_hbm.at[c, rt])

    return k(sidx3, didx3, jnp.zeros((rows_per_tile, 16), jnp.float32))


def _sc_scatter(m, sidx3, didx3):
    n, f = m.shape
    nw, iters, ch = sidx3.shape
    assert nw == _NW and ch == _CH and iters % _NBUF == 0
    rows_per_tile, n_pad = _row_pad(n)

    mesh = plsc.VectorSubcoreMesh(core_axis_name="c", subcore_axis_name="s")

    @functools.partial(
        pl.kernel,
        mesh=mesh,
        compiler_params=pltpu.CompilerParams(use_tc_tiling_on_sc=False),
        out_type=jax.ShapeDtypeStruct((2, n_pad, f), jnp.float32),
        scratch_types=[
            pltpu.VMEM((iters, _CH), jnp.int32),
            pltpu.VMEM((iters, _CH), jnp.int32),
        ] + [pltpu.VMEM((_CH, f), jnp.float32)] * _NBUF + [
            pltpu.VMEM_SHARED((n_pad, f), jnp.float32),
        ] + [pltpu.SemaphoreType.DMA] * _NBUF,
    )
    def k(m_hbm, src_hbm, dst_hbm, zeros_hbm, out_hbm, sidx, didx, *rest):
        rows = rest[:_NBUF]
        acc = rest[_NBUF]
        semg = rest[_NBUF + 1:2 * _NBUF + 1]
        c = lax.axis_index("c")
        s = lax.axis_index("s")
        wid = c * _NS + s
        base_rows = s * rows_per_tile

        pltpu.sync_copy(zeros_hbm, acc.at[pl.ds(base_rows, rows_per_tile)])
        pltpu.sync_copy(src_hbm.at[wid], sidx)
        pltpu.sync_copy(dst_hbm.at[wid], didx)
        plsc.subcore_barrier()

        def gather_start(i, b):
            pltpu.async_copy(m_hbm.at[sidx.at[i]], rows[b], semg[b])

        def gather_wait(i, b):
            pltpu.make_async_copy(m_hbm.at[sidx.at[i]], rows[b], semg[b]).wait()

        gather_start(0, 0)

        def grp(j, carry):
            i0 = j * 2
            gather_start(i0 + 1, 1)
            gather_wait(i0, 0)
            pltpu.sync_copy(rows[0], acc.at[didx.at[i0]], add=True)

            @pl.when(i0 + 2 < iters)
            def _():
                gather_start(i0 + 2, 0)

            gather_wait(i0 + 1, 1)
            pltpu.sync_copy(rows[1], acc.at[didx.at[i0 + 1]], add=True)
            return carry

        lax.fori_loop(0, iters // 2, grp, 0)

        plsc.subcore_barrier()
        rt = pl.ds(base_rows, rows_per_tile)
        pltpu.sync_copy(acc.at[rt], out_hbm.at[c, rt])

    return k(m, sidx3, didx3, jnp.zeros((rows_per_tile, f), jnp.float32))


def _onorm(degs_ref):
    od = degs_ref[0, :, 0:1] + degs_ref[1, :, 0:1]
    return lax.rsqrt(jnp.maximum(od, 1.0))


def _tc_a_body(x_ref, w_ref, degs_ref, m1_ref):
    m1_ref[...] = jnp.dot(
        x_ref[...] * _onorm(degs_ref), w_ref[...],
        preferred_element_type=jnp.float32)


def _tc_b_body(p1_ref, odeg_ref, ideg_ref, b1_ref, w2_ref, m2_ref):
    inorm = _onorm(ideg_ref)
    h = (p1_ref[0] + p1_ref[1]) * inorm + b1_ref[...]
    h = jnp.maximum(h, 0.0)
    m2_ref[...] = jnp.dot(
        h, w2_ref[...], preferred_element_type=jnp.float32) * _onorm(odeg_ref)


def _tc_c_body(p2_ref, ideg_ref, b2_ref, out_ref):
    z = (p2_ref[0] + p2_ref[1]) * _onorm(ideg_ref) + b2_ref[...]
    zmax = jnp.max(z, axis=-1, keepdims=True)
    zs = z - zmax
    out_ref[...] = zs - jnp.log(jnp.sum(jnp.exp(zs), axis=-1, keepdims=True))


def kernel(x, edge_index, W1, b1, W2, b2):
    n, nfeat = x.shape
    nhid = W1.shape[1]
    ncls = W2.shape[1]
    bn = 2000
    nblk = n // bn
    assert nblk * bn == n

    src = edge_index[0].astype(jnp.int32)
    dst = edge_index[1].astype(jnp.int32)
    e = src.shape[0]
    _, n_pad = _row_pad(n)

    iters = -(-e // (_NW * _CH * _NBUF)) * _NBUF
    e_pad = _NW * iters * _CH
    pk = jnp.arange(e_pad - e, dtype=jnp.int32)
    trash = n + pk % (n_pad - n)
    src_gat3 = jnp.concatenate([src, pk % n]).reshape(_NW, iters, _CH)
    src_cnt3 = jnp.concatenate([src, trash]).reshape(_NW, iters, _CH)
    dst3 = jnp.concatenate([dst, trash]).reshape(_NW, iters, _CH)

    odegs, idegs = _sc_bincount2(src_cnt3, dst3, n)

    odeg_spec = pl.BlockSpec((2, bn, 16), lambda i: (0, i, 0))
    ideg_spec = pl.BlockSpec((2, bn, 16), lambda i: (0, i, 0))

    m1 = pl.pallas_call(
        _tc_a_body,
        grid=(nblk,),
        in_specs=[
            pl.BlockSpec((bn, nfeat), lambda i: (i, 0)),
            pl.BlockSpec((nfeat, nhid), lambda i: (0, 0)),
            odeg_spec,
        ],
        out_specs=pl.BlockSpec((bn, nhid), lambda i: (i, 0)),
        out_shape=jax.ShapeDtypeStruct((n, nhid), jnp.float32),
    )(x, W1, odegs)

    part1 = _sc_scatter(m1, src_gat3, dst3)

    m2 = pl.pallas_call(
        _tc_b_body,
        grid=(nblk,),
        in_specs=[
            pl.BlockSpec((2, bn, nhid), lambda i: (0, i, 0)),
            odeg_spec,
            ideg_spec,
            pl.BlockSpec((1, nhid), lambda i: (0, 0)),
            pl.BlockSpec((nhid, ncls), lambda i: (0, 0)),
        ],
        out_specs=pl.BlockSpec((bn, ncls), lambda i: (i, 0)),
        out_shape=jax.ShapeDtypeStruct((n, ncls), jnp.float32),
    )(part1, odegs, idegs, b1.reshape(1, nhid), W2)

    part2 = _sc_scatter(m2, src_gat3, dst3)

    out = pl.pallas_call(
        _tc_c_body,
        grid=(nblk,),
        in_specs=[
            pl.BlockSpec((2, bn, ncls), lambda i: (0, i, 0)),
            ideg_spec,
            pl.BlockSpec((1, ncls), lambda i: (0, 0)),
        ],
        out_specs=pl.BlockSpec((bn, ncls), lambda i: (i, 0)),
        out_shape=jax.ShapeDtypeStruct((n, ncls), jnp.float32),
    )(part2, idegs, b2.reshape(1, ncls))

    return out


# speedup vs baseline: 1.1811x; 1.0077x over previous
import functools

import jax
import jax.numpy as jnp
from jax import lax
from jax.experimental import pallas as pl
from jax.experimental.pallas import tpu as pltpu
from jax.experimental.pallas import tpu_sc as plsc

_NC = 2
_NS = 16
_NW = _NC * _NS
_CH = 96
_NBUF = 2


def _row_pad(n):
    rows_per_tile = -(-n // (8 * _NS)) * 8
    return rows_per_tile, rows_per_tile * _NS


def _sc_bincount2(sidx3, didx3, n):
    nw, iters, ch = sidx3.shape
    assert nw == _NW and ch == _CH
    rows_per_tile, n_pad = _row_pad(n)

    mesh = plsc.VectorSubcoreMesh(core_axis_name="c", subcore_axis_name="s")

    @functools.partial(
        pl.kernel,
        mesh=mesh,
        compiler_params=pltpu.CompilerParams(use_tc_tiling_on_sc=False),
        out_type=[jax.ShapeDtypeStruct((2, n_pad, 16), jnp.float32),
                  jax.ShapeDtypeStruct((2, n_pad, 16), jnp.float32)],
        scratch_types=[
            pltpu.VMEM((iters, _CH), jnp.int32),
            pltpu.VMEM((iters, _CH), jnp.int32),
            pltpu.VMEM((_CH, 16), jnp.float32),
            pltpu.VMEM_SHARED((n_pad, 16), jnp.float32),
            pltpu.VMEM_SHARED((n_pad, 16), jnp.float32),
            pltpu.SemaphoreType.DMA,
            pltpu.SemaphoreType.DMA,
        ],
    )
    def k(sidx_hbm, didx_hbm, zeros_hbm, outs_hbm, outd_hbm, sidx, didx, ones,
          acc_s, acc_d, sem_s, sem_d):
        c = lax.axis_index("c")
        s = lax.axis_index("s")
        wid = c * _NS + s
        base_rows = s * rows_per_tile

        def fill_ones(i, carry):
            ones[i] = jnp.ones((16,), jnp.float32)
            return carry

        lax.fori_loop(0, _CH, fill_ones, 0)

        rt = pl.ds(base_rows, rows_per_tile)
        c1 = pltpu.async_copy(zeros_hbm, acc_s.at[rt], sem_s)
        c2 = pltpu.async_copy(zeros_hbm, acc_d.at[rt], sem_d)
        c3 = pltpu.async_copy(sidx_hbm.at[wid], sidx, sem_s)
        c4 = pltpu.async_copy(didx_hbm.at[wid], didx, sem_d)
        c1.wait(); c2.wait(); c3.wait(); c4.wait()
        plsc.subcore_barrier()

        def step(i, carry):
            pltpu.async_copy(ones, acc_s.at[sidx.at[i]], sem_s, add=True)
            pltpu.async_copy(ones, acc_d.at[didx.at[i]], sem_d, add=True)

            @pl.when(i > 0)
            def _():
                pltpu.make_async_copy(ones, acc_s.at[sidx.at[i]], sem_s).wait()
                pltpu.make_async_copy(ones, acc_d.at[didx.at[i]], sem_d).wait()

            return carry

        lax.fori_loop(0, iters, step, 0)
        pltpu.make_async_copy(ones, acc_s.at[sidx.at[0]], sem_s).wait()
        pltpu.make_async_copy(ones, acc_d.at[didx.at[0]], sem_d).wait()

        plsc.subcore_barrier()
        rt = pl.ds(base_rows, rows_per_tile)
        pltpu.sync_copy(acc_s.at[rt], outs_hbm.at[c, rt])
        pltpu.sync_copy(acc_d.at[rt], outd_hbm.at[c, rt])

    return k(sidx3, didx3, jnp.zeros((rows_per_tile, 16), jnp.float32))


def _sc_scatter(m, sidx3, didx3):
    n, f = m.shape
    nw, iters, ch = sidx3.shape
    assert nw == _NW and ch == _CH and iters % _NBUF == 0
    rows_per_tile, n_pad = _row_pad(n)

    mesh = plsc.VectorSubcoreMesh(core_axis_name="c", subcore_axis_name="s")

    @functools.partial(
        pl.kernel,
        mesh=mesh,
        compiler_params=pltpu.CompilerParams(use_tc_tiling_on_sc=False),
        out_type=jax.ShapeDtypeStruct((2, n_pad, f), jnp.float32),
        scratch_types=[
            pltpu.VMEM((iters, _CH), jnp.int32),
            pltpu.VMEM((iters, _CH), jnp.int32),
        ] + [pltpu.VMEM((_CH, f), jnp.float32)] * _NBUF + [
            pltpu.VMEM_SHARED((n_pad, f), jnp.float32),
        ] + [pltpu.SemaphoreType.DMA] * _NBUF,
    )
    def k(m_hbm, src_hbm, dst_hbm, zeros_hbm, out_hbm, sidx, didx, *rest):
        rows = rest[:_NBUF]
        acc = rest[_NBUF]
        semg = rest[_NBUF + 1:2 * _NBUF + 1]
        c = lax.axis_index("c")
        s = lax.axis_index("s")
        wid = c * _NS + s
        base_rows = s * rows_per_tile

        c1 = pltpu.async_copy(zeros_hbm, acc.at[pl.ds(base_rows,
                                                      rows_per_tile)], semg[0])
        c2 = pltpu.async_copy(src_hbm.at[wid], sidx, semg[0])
        c3 = pltpu.async_copy(dst_hbm.at[wid], didx, semg[1])
        c1.wait(); c2.wait(); c3.wait()
        plsc.subcore_barrier()

        def gather_start(i, b):
            pltpu.async_copy(m_hbm.at[sidx.at[i]], rows[b], semg[b])

        def gather_wait(i, b):
            pltpu.make_async_copy(m_hbm.at[sidx.at[i]], rows[b], semg[b]).wait()

        gather_start(0, 0)

        def grp(j, carry):
            i0 = j * 2
            gather_start(i0 + 1, 1)
            gather_wait(i0, 0)
            pltpu.sync_copy(rows[0], acc.at[didx.at[i0]], add=True)

            @pl.when(i0 + 2 < iters)
            def _():
                gather_start(i0 + 2, 0)

            gather_wait(i0 + 1, 1)
            pltpu.sync_copy(rows[1], acc.at[didx.at[i0 + 1]], add=True)
            return carry

        lax.fori_loop(0, iters // 2, grp, 0)

        plsc.subcore_barrier()
        rt = pl.ds(base_rows, rows_per_tile)
        pltpu.sync_copy(acc.at[rt], out_hbm.at[c, rt])

    return k(m, sidx3, didx3, jnp.zeros((rows_per_tile, f), jnp.float32))


def _onorm(degs_ref):
    od = degs_ref[0, :, 0:1] + degs_ref[1, :, 0:1]
    return lax.rsqrt(jnp.maximum(od, 1.0))


def _tc_mm_body(x_ref, w_ref, xw_ref):
    xw_ref[...] = jnp.dot(x_ref[...], w_ref[...],
                          preferred_element_type=jnp.float32)


def _tc_scale_body(xw_ref, degs_ref, m1_ref):
    m1_ref[...] = xw_ref[...] * _onorm(degs_ref)


def _tc_b_body(p1_ref, odeg_ref, ideg_ref, b1_ref, w2_ref, m2_ref):
    inorm = _onorm(ideg_ref)
    h = (p1_ref[0] + p1_ref[1]) * inorm + b1_ref[...]
    h = jnp.maximum(h, 0.0)
    m2_ref[...] = jnp.dot(
        h, w2_ref[...], preferred_element_type=jnp.float32) * _onorm(odeg_ref)


def _tc_c_body(p2_ref, ideg_ref, b2_ref, out_ref):
    z = (p2_ref[0] + p2_ref[1]) * _onorm(ideg_ref) + b2_ref[...]
    zmax = jnp.max(z, axis=-1, keepdims=True)
    zs = z - zmax
    out_ref[...] = zs - jnp.log(jnp.sum(jnp.exp(zs), axis=-1, keepdims=True))


def kernel(x, edge_index, W1, b1, W2, b2):
    n, nfeat = x.shape
    nhid = W1.shape[1]
    ncls = W2.shape[1]
    bn = 2000
    nblk = n // bn
    assert nblk * bn == n

    src = edge_index[0].astype(jnp.int32)
    dst = edge_index[1].astype(jnp.int32)
    e = src.shape[0]
    _, n_pad = _row_pad(n)

    iters = -(-e // (_NW * _CH * _NBUF)) * _NBUF
    e_pad = _NW * iters * _CH
    pk = jnp.arange(e_pad - e, dtype=jnp.int32)
    trash = n + pk % (n_pad - n)
    src_gat3 = jnp.concatenate([src, pk % n]).reshape(_NW, iters, _CH)
    src_cnt3 = jnp.concatenate([src, trash]).reshape(_NW, iters, _CH)
    dst3 = jnp.concatenate([dst, trash]).reshape(_NW, iters, _CH)

    odegs, idegs = _sc_bincount2(src_cnt3, dst3, n)

    odeg_spec = pl.BlockSpec((2, bn, 16), lambda i: (0, i, 0))
    ideg_spec = pl.BlockSpec((2, bn, 16), lambda i: (0, i, 0))

    xw1 = pl.pallas_call(
        _tc_mm_body,
        grid=(nblk,),
        in_specs=[
            pl.BlockSpec((bn, nfeat), lambda i: (i, 0)),
            pl.BlockSpec((nfeat, nhid), lambda i: (0, 0)),
        ],
        out_specs=pl.BlockSpec((bn, nhid), lambda i: (i, 0)),
        out_shape=jax.ShapeDtypeStruct((n, nhid), jnp.float32),
    )(x, W1)

    m1 = pl.pallas_call(
        _tc_scale_body,
        grid=(nblk,),
        in_specs=[
            pl.BlockSpec((bn, nhid), lambda i: (i, 0)),
            odeg_spec,
        ],
        out_specs=pl.BlockSpec((bn, nhid), lambda i: (i, 0)),
        out_shape=jax.ShapeDtypeStruct((n, nhid), jnp.float32),
    )(xw1, odegs)

    part1 = _sc_scatter(m1, src_gat3, dst3)

    m2 = pl.pallas_call(
        _tc_b_body,
        grid=(nblk,),
        in_specs=[
            pl.BlockSpec((2, bn, nhid), lambda i: (0, i, 0)),
            odeg_spec,
            ideg_spec,
            pl.BlockSpec((1, nhid), lambda i: (0, 0)),
            pl.BlockSpec((nhid, ncls), lambda i: (0, 0)),
        ],
        out_specs=pl.BlockSpec((bn, ncls), lambda i: (i, 0)),
        out_shape=jax.ShapeDtypeStruct((n, ncls), jnp.float32),
    )(part1, odegs, idegs, b1.reshape(1, nhid), W2)

    part2 = _sc_scatter(m2, src_gat3, dst3)

    out = pl.pallas_call(
        _tc_c_body,
        grid=(nblk,),
        in_specs=[
            pl.BlockSpec((2, bn, ncls), lambda i: (0, i, 0)),
            ideg_spec,
            pl.BlockSpec((1, ncls), lambda i: (0, 0)),
        ],
        out_specs=pl.BlockSpec((bn, ncls), lambda i: (i, 0)),
        out_shape=jax.ShapeDtypeStruct((n, ncls), jnp.float32),
    )(part2, idegs, b2.reshape(1, ncls))

    return out


# speedup vs baseline: 1.2196x; 1.0325x over previous
import functools

import jax
import jax.numpy as jnp
from jax import lax
from jax.experimental import pallas as pl
from jax.experimental.pallas import tpu as pltpu
from jax.experimental.pallas import tpu_sc as plsc

_NC = 2
_NS = 16
_NW = _NC * _NS
_CH = 112
_NBUF = 2


def _row_pad(n):
    rows_per_tile = -(-n // (8 * _NS)) * 8
    return rows_per_tile, rows_per_tile * _NS


def _sc_bincount2(sidx3, didx3, n):
    nw, iters, ch = sidx3.shape
    assert nw == _NW and ch == _CH
    rows_per_tile, n_pad = _row_pad(n)

    mesh = plsc.VectorSubcoreMesh(core_axis_name="c", subcore_axis_name="s")

    @functools.partial(
        pl.kernel,
        mesh=mesh,
        compiler_params=pltpu.CompilerParams(use_tc_tiling_on_sc=False),
        out_type=[jax.ShapeDtypeStruct((2, n_pad, 16), jnp.float32),
                  jax.ShapeDtypeStruct((2, n_pad, 16), jnp.float32)],
        scratch_types=[
            pltpu.VMEM((iters, _CH), jnp.int32),
            pltpu.VMEM((iters, _CH), jnp.int32),
            pltpu.VMEM((_CH, 16), jnp.float32),
            pltpu.VMEM_SHARED((n_pad, 16), jnp.float32),
            pltpu.VMEM_SHARED((n_pad, 16), jnp.float32),
            pltpu.SemaphoreType.DMA,
            pltpu.SemaphoreType.DMA,
        ],
    )
    def k(sidx_hbm, didx_hbm, zeros_hbm, outs_hbm, outd_hbm, sidx, didx, ones,
          acc_s, acc_d, sem_s, sem_d):
        c = lax.axis_index("c")
        s = lax.axis_index("s")
        wid = c * _NS + s
        base_rows = s * rows_per_tile

        def fill_ones(i, carry):
            ones[i] = jnp.ones((16,), jnp.float32)
            return carry

        lax.fori_loop(0, _CH, fill_ones, 0)

        rt = pl.ds(base_rows, rows_per_tile)
        c1 = pltpu.async_copy(zeros_hbm, acc_s.at[rt], sem_s)
        c2 = pltpu.async_copy(zeros_hbm, acc_d.at[rt], sem_d)
        c3 = pltpu.async_copy(sidx_hbm.at[wid], sidx, sem_s)
        c4 = pltpu.async_copy(didx_hbm.at[wid], didx, sem_d)
        c1.wait(); c2.wait(); c3.wait(); c4.wait()
        plsc.subcore_barrier()

        def step(i, carry):
            pltpu.async_copy(ones, acc_s.at[sidx.at[i]], sem_s, add=True)
            pltpu.async_copy(ones, acc_d.at[didx.at[i]], sem_d, add=True)

            @pl.when(i > 0)
            def _():
                pltpu.make_async_copy(ones, acc_s.at[sidx.at[i]], sem_s).wait()
                pltpu.make_async_copy(ones, acc_d.at[didx.at[i]], sem_d).wait()

            return carry

        lax.fori_loop(0, iters, step, 0)
        pltpu.make_async_copy(ones, acc_s.at[sidx.at[0]], sem_s).wait()
        pltpu.make_async_copy(ones, acc_d.at[didx.at[0]], sem_d).wait()

        plsc.subcore_barrier()
        rt = pl.ds(base_rows, rows_per_tile)
        pltpu.sync_copy(acc_s.at[rt], outs_hbm.at[c, rt])
        pltpu.sync_copy(acc_d.at[rt], outd_hbm.at[c, rt])

    return k(sidx3, didx3, jnp.zeros((rows_per_tile, 16), jnp.float32))


def _sc_scatter(m, sidx3, didx3):
    n, f = m.shape
    nw, iters, ch = sidx3.shape
    assert nw == _NW and ch == _CH and iters % _NBUF == 0
    rows_per_tile, n_pad = _row_pad(n)

    mesh = plsc.VectorSubcoreMesh(core_axis_name="c", subcore_axis_name="s")

    @functools.partial(
        pl.kernel,
        mesh=mesh,
        compiler_params=pltpu.CompilerParams(use_tc_tiling_on_sc=False),
        out_type=jax.ShapeDtypeStruct((2, n_pad, f), jnp.float32),
        scratch_types=[
            pltpu.VMEM((iters, _CH), jnp.int32),
            pltpu.VMEM((iters, _CH), jnp.int32),
        ] + [pltpu.VMEM((_CH, f), jnp.float32)] * _NBUF + [
            pltpu.VMEM_SHARED((n_pad, f), jnp.float32),
        ] + [pltpu.SemaphoreType.DMA] * _NBUF,
    )
    def k(m_hbm, src_hbm, dst_hbm, zeros_hbm, out_hbm, sidx, didx, *rest):
        rows = rest[:_NBUF]
        acc = rest[_NBUF]
        semg = rest[_NBUF + 1:2 * _NBUF + 1]
        c = lax.axis_index("c")
        s = lax.axis_index("s")
        wid = c * _NS + s
        base_rows = s * rows_per_tile

        c1 = pltpu.async_copy(zeros_hbm, acc.at[pl.ds(base_rows,
                                                      rows_per_tile)], semg[0])
        c2 = pltpu.async_copy(src_hbm.at[wid], sidx, semg[0])
        c3 = pltpu.async_copy(dst_hbm.at[wid], didx, semg[1])
        c1.wait(); c2.wait(); c3.wait()
        plsc.subcore_barrier()

        def gather_start(i, b):
            pltpu.async_copy(m_hbm.at[sidx.at[i]], rows[b], semg[b])

        def gather_wait(i, b):
            pltpu.make_async_copy(m_hbm.at[sidx.at[i]], rows[b], semg[b]).wait()

        gather_start(0, 0)

        def grp(j, carry):
            i0 = j * 2
            gather_start(i0 + 1, 1)
            gather_wait(i0, 0)
            pltpu.sync_copy(rows[0], acc.at[didx.at[i0]], add=True)

            @pl.when(i0 + 2 < iters)
            def _():
                gather_start(i0 + 2, 0)

            gather_wait(i0 + 1, 1)
            pltpu.sync_copy(rows[1], acc.at[didx.at[i0 + 1]], add=True)
            return carry

        lax.fori_loop(0, iters // 2, grp, 0)

        plsc.subcore_barrier()
        rt = pl.ds(base_rows, rows_per_tile)
        pltpu.sync_copy(acc.at[rt], out_hbm.at[c, rt])

    return k(m, sidx3, didx3, jnp.zeros((rows_per_tile, f), jnp.float32))


def _onorm(degs_ref):
    od = degs_ref[0, :, 0:1] + degs_ref[1, :, 0:1]
    return lax.rsqrt(jnp.maximum(od, 1.0))


def _tc_mm_body(x_ref, w_ref, xw_ref):
    xw_ref[...] = jnp.dot(x_ref[...], w_ref[...],
                          preferred_element_type=jnp.float32)


def _tc_scale_body(xw_ref, degs_ref, m1_ref):
    m1_ref[...] = xw_ref[...] * _onorm(degs_ref)


def _tc_b_body(p1_ref, odeg_ref, ideg_ref, b1_ref, w2_ref, m2_ref):
    inorm = _onorm(ideg_ref)
    h = (p1_ref[0] + p1_ref[1]) * inorm + b1_ref[...]
    h = jnp.maximum(h, 0.0)
    m2_ref[...] = jnp.dot(
        h, w2_ref[...], preferred_element_type=jnp.float32) * _onorm(odeg_ref)


def _tc_c_body(p2_ref, ideg_ref, b2_ref, out_ref):
    z = (p2_ref[0] + p2_ref[1]) * _onorm(ideg_ref) + b2_ref[...]
    zmax = jnp.max(z, axis=-1, keepdims=True)
    zs = z - zmax
    out_ref[...] = zs - jnp.log(jnp.sum(jnp.exp(zs), axis=-1, keepdims=True))


def kernel(x, edge_index, W1, b1, W2, b2):
    n, nfeat = x.shape
    nhid = W1.shape[1]
    ncls = W2.shape[1]
    bn = 2000
    nblk = n // bn
    assert nblk * bn == n

    src = edge_index[0].astype(jnp.int32)
    dst = edge_index[1].astype(jnp.int32)
    e = src.shape[0]
    _, n_pad = _row_pad(n)

    iters = -(-e // (_NW * _CH * _NBUF)) * _NBUF
    e_pad = _NW * iters * _CH
    pk = jnp.arange(e_pad - e, dtype=jnp.int32)
    trash = n + pk % (n_pad - n)
    src_gat3 = jnp.concatenate([src, pk % n]).reshape(_NW, iters, _CH)
    src_cnt3 = jnp.concatenate([src, trash]).reshape(_NW, iters, _CH)
    dst3 = jnp.concatenate([dst, trash]).reshape(_NW, iters, _CH)

    odegs, idegs = _sc_bincount2(src_cnt3, dst3, n)

    odeg_spec = pl.BlockSpec((2, bn, 16), lambda i: (0, i, 0))
    ideg_spec = pl.BlockSpec((2, bn, 16), lambda i: (0, i, 0))

    xw1 = pl.pallas_call(
        _tc_mm_body,
        grid=(nblk,),
        in_specs=[
            pl.BlockSpec((bn, nfeat), lambda i: (i, 0)),
            pl.BlockSpec((nfeat, nhid), lambda i: (0, 0)),
        ],
        out_specs=pl.BlockSpec((bn, nhid), lambda i: (i, 0)),
        out_shape=jax.ShapeDtypeStruct((n, nhid), jnp.float32),
    )(x, W1)

    m1 = pl.pallas_call(
        _tc_scale_body,
        grid=(nblk,),
        in_specs=[
            pl.BlockSpec((bn, nhid), lambda i: (i, 0)),
            odeg_spec,
        ],
        out_specs=pl.BlockSpec((bn, nhid), lambda i: (i, 0)),
        out_shape=jax.ShapeDtypeStruct((n, nhid), jnp.float32),
    )(xw1, odegs)

    part1 = _sc_scatter(m1, src_gat3, dst3)

    m2 = pl.pallas_call(
        _tc_b_body,
        grid=(nblk,),
        in_specs=[
            pl.BlockSpec((2, bn, nhid), lambda i: (0, i, 0)),
            odeg_spec,
            ideg_spec,
            pl.BlockSpec((1, nhid), lambda i: (0, 0)),
            pl.BlockSpec((nhid, ncls), lambda i: (0, 0)),
        ],
        out_specs=pl.BlockSpec((bn, ncls), lambda i: (i, 0)),
        out_shape=jax.ShapeDtypeStruct((n, ncls), jnp.float32),
    )(part1, odegs, idegs, b1.reshape(1, nhid), W2)

    part2 = _sc_scatter(m2, src_gat3, dst3)

    out = pl.pallas_call(
        _tc_c_body,
        grid=(nblk,),
        in_specs=[
            pl.BlockSpec((2, bn, ncls), lambda i: (0, i, 0)),
            ideg_spec,
            pl.BlockSpec((1, ncls), lambda i: (0, 0)),
        ],
        out_specs=pl.BlockSpec((bn, ncls), lambda i: (i, 0)),
        out_shape=jax.ShapeDtypeStruct((n, ncls), jnp.float32),
    )(part2, idegs, b2.reshape(1, ncls))

    return out


# speedup vs baseline: 1.2272x; 1.0063x over previous
import functools

import jax
import jax.numpy as jnp
from jax import lax
from jax.experimental import pallas as pl
from jax.experimental.pallas import tpu as pltpu
from jax.experimental.pallas import tpu_sc as plsc

_NC = 2
_NS = 16
_NW = _NC * _NS
_CH = 112
_NBUF = 2


def _row_pad(n):
    rows_per_tile = -(-n // (8 * _NS)) * 8
    return rows_per_tile, rows_per_tile * _NS


def _sc_bincount2(sidx3, didx3, n):
    nw, iters, ch = sidx3.shape
    assert nw == _NW and ch == _CH
    rows_per_tile, n_pad = _row_pad(n)

    mesh = plsc.VectorSubcoreMesh(core_axis_name="c", subcore_axis_name="s")

    @functools.partial(
        pl.kernel,
        mesh=mesh,
        compiler_params=pltpu.CompilerParams(use_tc_tiling_on_sc=False),
        out_type=[jax.ShapeDtypeStruct((2, n_pad, 16), jnp.float32),
                  jax.ShapeDtypeStruct((2, n_pad, 16), jnp.float32)],
        scratch_types=[
            pltpu.VMEM((iters, _CH), jnp.int32),
            pltpu.VMEM((iters, _CH), jnp.int32),
            pltpu.VMEM((_CH, 16), jnp.float32),
            pltpu.VMEM_SHARED((n_pad, 16), jnp.float32),
            pltpu.VMEM_SHARED((n_pad, 16), jnp.float32),
            pltpu.SemaphoreType.DMA,
            pltpu.SemaphoreType.DMA,
        ],
    )
    def k(sidx_hbm, didx_hbm, zeros_hbm, outs_hbm, outd_hbm, sidx, didx, ones,
          acc_s, acc_d, sem_s, sem_d):
        c = lax.axis_index("c")
        s = lax.axis_index("s")
        wid = c * _NS + s
        base_rows = s * rows_per_tile

        def fill_ones(i, carry):
            ones[i] = jnp.ones((16,), jnp.float32)
            return carry

        lax.fori_loop(0, _CH, fill_ones, 0)

        rt = pl.ds(base_rows, rows_per_tile)
        c1 = pltpu.async_copy(zeros_hbm, acc_s.at[rt], sem_s)
        c2 = pltpu.async_copy(zeros_hbm, acc_d.at[rt], sem_d)
        c3 = pltpu.async_copy(sidx_hbm.at[wid], sidx, sem_s)
        c4 = pltpu.async_copy(didx_hbm.at[wid], didx, sem_d)
        c1.wait(); c2.wait(); c3.wait(); c4.wait()
        plsc.subcore_barrier()

        def step(i, carry):
            pltpu.async_copy(ones, acc_s.at[sidx.at[i]], sem_s, add=True)
            pltpu.async_copy(ones, acc_d.at[didx.at[i]], sem_d, add=True)

            @pl.when(i > 0)
            def _():
                pltpu.make_async_copy(ones, acc_s.at[sidx.at[i]], sem_s).wait()
                pltpu.make_async_copy(ones, acc_d.at[didx.at[i]], sem_d).wait()

            return carry

        lax.fori_loop(0, iters, step, 0)
        pltpu.make_async_copy(ones, acc_s.at[sidx.at[0]], sem_s).wait()
        pltpu.make_async_copy(ones, acc_d.at[didx.at[0]], sem_d).wait()

        plsc.subcore_barrier()
        rt = pl.ds(base_rows, rows_per_tile)
        pltpu.sync_copy(acc_s.at[rt], outs_hbm.at[c, rt])
        pltpu.sync_copy(acc_d.at[rt], outd_hbm.at[c, rt])

    return k(sidx3, didx3, jnp.zeros((rows_per_tile, 16), jnp.float32))


def _sc_scatter(m, sidx3, didx3):
    n, f = m.shape
    nw, iters, ch = sidx3.shape
    assert nw == _NW and ch == _CH and iters % _NBUF == 0
    rows_per_tile, n_pad = _row_pad(n)

    mesh = plsc.VectorSubcoreMesh(core_axis_name="c", subcore_axis_name="s")

    @functools.partial(
        pl.kernel,
        mesh=mesh,
        compiler_params=pltpu.CompilerParams(use_tc_tiling_on_sc=False),
        out_type=jax.ShapeDtypeStruct((2, n_pad, f), jnp.float32),
        scratch_types=[
            pltpu.VMEM((iters, _CH), jnp.int32),
            pltpu.VMEM((iters, _CH), jnp.int32),
        ] + [pltpu.VMEM((_CH, f), jnp.float32)] * _NBUF + [
            pltpu.VMEM_SHARED((n_pad, f), jnp.float32),
        ] + [pltpu.SemaphoreType.DMA] * _NBUF,
    )
    def k(m_hbm, src_hbm, dst_hbm, zeros_hbm, out_hbm, sidx, didx, *rest):
        rows = rest[:_NBUF]
        acc = rest[_NBUF]
        semg = rest[_NBUF + 1:2 * _NBUF + 1]
        c = lax.axis_index("c")
        s = lax.axis_index("s")
        wid = c * _NS + s
        base_rows = s * rows_per_tile

        c1 = pltpu.async_copy(zeros_hbm, acc.at[pl.ds(base_rows,
                                                      rows_per_tile)], semg[0])
        c2 = pltpu.async_copy(src_hbm.at[wid], sidx, semg[0])
        c3 = pltpu.async_copy(dst_hbm.at[wid], didx, semg[1])
        c1.wait(); c2.wait(); c3.wait()
        plsc.subcore_barrier()

        def gather_start(i, b):
            pltpu.async_copy(m_hbm.at[sidx.at[i]], rows[b], semg[b])

        def gather_wait(i, b):
            pltpu.make_async_copy(m_hbm.at[sidx.at[i]], rows[b], semg[b]).wait()

        gather_start(0, 0)

        def grp(j, carry):
            i0 = j * 2
            gather_start(i0 + 1, 1)
            gather_wait(i0, 0)
            pltpu.sync_copy(rows[0], acc.at[didx.at[i0]], add=True)

            @pl.when(i0 + 2 < iters)
            def _():
                gather_start(i0 + 2, 0)

            gather_wait(i0 + 1, 1)
            pltpu.sync_copy(rows[1], acc.at[didx.at[i0 + 1]], add=True)
            return carry

        lax.fori_loop(0, iters // 2, grp, 0)

        plsc.subcore_barrier()
        rt = pl.ds(base_rows, rows_per_tile)
        pltpu.sync_copy(acc.at[rt], out_hbm.at[c, rt])

    return k(m, sidx3, didx3, jnp.zeros((rows_per_tile, f), jnp.float32))


def _onorm(degs_ref):
    od = degs_ref[0, :, 0:1] + degs_ref[1, :, 0:1]
    return lax.rsqrt(jnp.maximum(od, 1.0))


def _tc_mm_body(x_ref, w_ref, xw_ref):
    xw_ref[...] = jnp.dot(x_ref[...], w_ref[...],
                          preferred_element_type=jnp.float32)


def _tc_scale_body(xw_ref, degs_ref, m1_ref):
    m1_ref[...] = xw_ref[...] * _onorm(degs_ref)


def _tc_a_body(x_ref, w_ref, degs_ref, m1_ref):
    m1_ref[...] = jnp.dot(
        x_ref[...] * _onorm(degs_ref), w_ref[...],
        preferred_element_type=jnp.float32)


def _tc_b_body(p1_ref, odeg_ref, ideg_ref, b1_ref, w2_ref, m2_ref):
    inorm = _onorm(ideg_ref)
    h = (p1_ref[0] + p1_ref[1]) * inorm + b1_ref[...]
    h = jnp.maximum(h, 0.0)
    m2_ref[...] = jnp.dot(
        h, w2_ref[...], preferred_element_type=jnp.float32) * _onorm(odeg_ref)


def _tc_c_body(p2_ref, ideg_ref, b2_ref, out_ref):
    z = (p2_ref[0] + p2_ref[1]) * _onorm(ideg_ref) + b2_ref[...]
    zmax = jnp.max(z, axis=-1, keepdims=True)
    zs = z - zmax
    out_ref[...] = zs - jnp.log(jnp.sum(jnp.exp(zs), axis=-1, keepdims=True))


def kernel(x, edge_index, W1, b1, W2, b2):
    n, nfeat = x.shape
    nhid = W1.shape[1]
    ncls = W2.shape[1]
    bn = 2000
    nblk = n // bn
    assert nblk * bn == n

    src = edge_index[0].astype(jnp.int32)
    dst = edge_index[1].astype(jnp.int32)
    e = src.shape[0]
    _, n_pad = _row_pad(n)

    iters = -(-e // (_NW * _CH * _NBUF)) * _NBUF
    e_pad = _NW * iters * _CH
    pk = jnp.arange(e_pad - e, dtype=jnp.int32)
    trash = n + pk % (n_pad - n)
    src_gat3 = jnp.concatenate([src, pk % n]).reshape(_NW, iters, _CH)
    src_cnt3 = jnp.concatenate([src, trash]).reshape(_NW, iters, _CH)
    dst3 = jnp.concatenate([dst, trash]).reshape(_NW, iters, _CH)

    odegs, idegs = _sc_bincount2(src_cnt3, dst3, n)

    odeg_spec = pl.BlockSpec((2, bn, 16), lambda i: (0, i, 0))
    ideg_spec = pl.BlockSpec((2, bn, 16), lambda i: (0, i, 0))

    m1 = pl.pallas_call(
        _tc_a_body,
        grid=(nblk,),
        in_specs=[
            pl.BlockSpec((bn, nfeat), lambda i: (i, 0)),
            pl.BlockSpec((nfeat, nhid), lambda i: (0, 0)),
            odeg_spec,
        ],
        out_specs=pl.BlockSpec((bn, nhid), lambda i: (i, 0)),
        out_shape=jax.ShapeDtypeStruct((n, nhid), jnp.float32),
    )(x, W1, odegs)

    part1 = _sc_scatter(m1, src_gat3, dst3)

    m2 = pl.pallas_call(
        _tc_b_body,
        grid=(nblk,),
        in_specs=[
            pl.BlockSpec((2, bn, nhid), lambda i: (0, i, 0)),
            odeg_spec,
            ideg_spec,
            pl.BlockSpec((1, nhid), lambda i: (0, 0)),
            pl.BlockSpec((nhid, ncls), lambda i: (0, 0)),
        ],
        out_specs=pl.BlockSpec((bn, ncls), lambda i: (i, 0)),
        out_shape=jax.ShapeDtypeStruct((n, ncls), jnp.float32),
    )(part1, odegs, idegs, b1.reshape(1, nhid), W2)

    part2 = _sc_scatter(m2, src_gat3, dst3)

    out = pl.pallas_call(
        _tc_c_body,
        grid=(nblk,),
        in_specs=[
            pl.BlockSpec((2, bn, ncls), lambda i: (0, i, 0)),
            ideg_spec,
            pl.BlockSpec((1, ncls), lambda i: (0, 0)),
        ],
        out_specs=pl.BlockSpec((bn, ncls), lambda i: (i, 0)),
        out_shape=jax.ShapeDtypeStruct((n, ncls), jnp.float32),
    )(part2, idegs, b2.reshape(1, ncls))

    return out


# speedup vs baseline: 1.2940x; 1.0544x over previous
import functools

import jax
import jax.numpy as jnp
from jax import lax
from jax.experimental import pallas as pl
from jax.experimental.pallas import tpu as pltpu
from jax.experimental.pallas import tpu_sc as plsc

_NC = 2
_NS = 16
_NW = _NC * _NS
_CH = 112
_NBUF = 2


def _row_pad(n):
    rows_per_tile = -(-n // (8 * _NS)) * 8
    return rows_per_tile, rows_per_tile * _NS


def _sc_bincount2(sidx3, didx3, n):
    nw, iters, ch = sidx3.shape
    assert nw == _NW and ch == _CH
    rows_per_tile, n_pad = _row_pad(n)

    mesh = plsc.VectorSubcoreMesh(core_axis_name="c", subcore_axis_name="s")

    @functools.partial(
        pl.kernel,
        mesh=mesh,
        compiler_params=pltpu.CompilerParams(use_tc_tiling_on_sc=False),
        out_type=[jax.ShapeDtypeStruct((2, n_pad, 16), jnp.float32),
                  jax.ShapeDtypeStruct((2, n_pad, 16), jnp.float32)],
        scratch_types=[
            pltpu.VMEM((iters, _CH), jnp.int32),
            pltpu.VMEM((iters, _CH), jnp.int32),
            pltpu.VMEM((_CH, 16), jnp.float32),
            pltpu.VMEM_SHARED((n_pad, 16), jnp.float32),
            pltpu.VMEM_SHARED((n_pad, 16), jnp.float32),
            pltpu.SemaphoreType.DMA,
            pltpu.SemaphoreType.DMA,
        ],
    )
    def k(sidx_hbm, didx_hbm, zeros_hbm, outs_hbm, outd_hbm, sidx, didx, ones,
          acc_s, acc_d, sem_s, sem_d):
        c = lax.axis_index("c")
        s = lax.axis_index("s")
        wid = c * _NS + s
        base_rows = s * rows_per_tile

        def fill_ones(i, carry):
            ones[i] = jnp.ones((16,), jnp.float32)
            return carry

        lax.fori_loop(0, _CH, fill_ones, 0)

        rt = pl.ds(base_rows, rows_per_tile)
        c1 = pltpu.async_copy(zeros_hbm, acc_s.at[rt], sem_s)
        c2 = pltpu.async_copy(zeros_hbm, acc_d.at[rt], sem_d)
        c3 = pltpu.async_copy(sidx_hbm.at[wid], sidx, sem_s)
        c4 = pltpu.async_copy(didx_hbm.at[wid], didx, sem_d)
        c1.wait(); c2.wait(); c3.wait(); c4.wait()
        plsc.subcore_barrier()

        def step(i, carry):
            pltpu.async_copy(ones, acc_s.at[sidx.at[i]], sem_s, add=True)
            pltpu.async_copy(ones, acc_d.at[didx.at[i]], sem_d, add=True)

            @pl.when(i > 0)
            def _():
                pltpu.make_async_copy(ones, acc_s.at[sidx.at[i]], sem_s).wait()
                pltpu.make_async_copy(ones, acc_d.at[didx.at[i]], sem_d).wait()

            return carry

        lax.fori_loop(0, iters, step, 0)
        pltpu.make_async_copy(ones, acc_s.at[sidx.at[0]], sem_s).wait()
        pltpu.make_async_copy(ones, acc_d.at[didx.at[0]], sem_d).wait()

        plsc.subcore_barrier()
        rt = pl.ds(base_rows, rows_per_tile)
        pltpu.sync_copy(acc_s.at[rt], outs_hbm.at[c, rt])
        pltpu.sync_copy(acc_d.at[rt], outd_hbm.at[c, rt])

    return k(sidx3, didx3, jnp.zeros((rows_per_tile, 16), jnp.float32))


def _sc_scatter(m, sidx3, didx3):
    n, f = m.shape
    nw, iters, ch = sidx3.shape
    rows_per_tile, n_pad = _row_pad(n)
    nbuf = _NBUF
    for cand in (4, 3):
        words = 16 * (2 * iters * _CH + cand * _CH * f) + n_pad * f
        if iters % cand == 0 and words <= 1_900_000:
            nbuf = cand
            break
    assert nw == _NW and ch == _CH and iters % nbuf == 0

    mesh = plsc.VectorSubcoreMesh(core_axis_name="c", subcore_axis_name="s")

    @functools.partial(
        pl.kernel,
        mesh=mesh,
        compiler_params=pltpu.CompilerParams(use_tc_tiling_on_sc=False),
        out_type=jax.ShapeDtypeStruct((2, n_pad, f), jnp.float32),
        scratch_types=[
            pltpu.VMEM((iters, _CH), jnp.int32),
            pltpu.VMEM((iters, _CH), jnp.int32),
        ] + [pltpu.VMEM((_CH, f), jnp.float32)] * nbuf + [
            pltpu.VMEM_SHARED((n_pad, f), jnp.float32),
        ] + [pltpu.SemaphoreType.DMA] * nbuf,
    )
    def k(m_hbm, src_hbm, dst_hbm, zeros_hbm, out_hbm, sidx, didx, *rest):
        rows = rest[:nbuf]
        acc = rest[nbuf]
        semg = rest[nbuf + 1:2 * nbuf + 1]
        c = lax.axis_index("c")
        s = lax.axis_index("s")
        wid = c * _NS + s
        base_rows = s * rows_per_tile

        c1 = pltpu.async_copy(zeros_hbm, acc.at[pl.ds(base_rows,
                                                      rows_per_tile)], semg[0])
        c2 = pltpu.async_copy(src_hbm.at[wid], sidx, semg[0])
        c3 = pltpu.async_copy(dst_hbm.at[wid], didx, semg[1])
        c1.wait(); c2.wait(); c3.wait()
        plsc.subcore_barrier()

        def gather_start(i, b):
            pltpu.async_copy(m_hbm.at[sidx.at[i]], rows[b], semg[b])

        def gather_wait(i, b):
            pltpu.make_async_copy(m_hbm.at[sidx.at[i]], rows[b], semg[b]).wait()

        for b in range(nbuf):
            gather_start(b, b)

        def grp(j, carry):
            i0 = j * nbuf
            for b in range(nbuf):
                gather_wait(i0 + b, b)
                pltpu.sync_copy(rows[b], acc.at[didx.at[i0 + b]], add=True)

                @pl.when(i0 + b + nbuf < iters)
                def _(b=b):
                    gather_start(i0 + b + nbuf, b)

            return carry

        lax.fori_loop(0, iters // nbuf, grp, 0)

        plsc.subcore_barrier()
        rt = pl.ds(base_rows, rows_per_tile)
        pltpu.sync_copy(acc.at[rt], out_hbm.at[c, rt])

    return k(m, sidx3, didx3, jnp.zeros((rows_per_tile, f), jnp.float32))


def _onorm(degs_ref):
    od = degs_ref[0, :, 0:1] + degs_ref[1, :, 0:1]
    return lax.rsqrt(jnp.maximum(od, 1.0))


def _tc_mm_body(x_ref, w_ref, xw_ref):
    xw_ref[...] = jnp.dot(x_ref[...], w_ref[...],
                          preferred_element_type=jnp.float32)


def _tc_scale_body(xw_ref, degs_ref, m1_ref):
    m1_ref[...] = xw_ref[...] * _onorm(degs_ref)


def _tc_a_body(x_ref, w_ref, degs_ref, m1_ref):
    m1_ref[...] = jnp.dot(
        x_ref[...] * _onorm(degs_ref), w_ref[...],
        preferred_element_type=jnp.float32)


def _tc_b_body(p1_ref, odeg_ref, ideg_ref, b1_ref, w2_ref, m2_ref):
    inorm = _onorm(ideg_ref)
    h = (p1_ref[0] + p1_ref[1]) * inorm + b1_ref[...]
    h = jnp.maximum(h, 0.0)
    m2_ref[...] = jnp.dot(
        h, w2_ref[...], preferred_element_type=jnp.float32) * _onorm(odeg_ref)


def _tc_c_body(p2_ref, ideg_ref, b2_ref, out_ref):
    z = (p2_ref[0] + p2_ref[1]) * _onorm(ideg_ref) + b2_ref[...]
    zmax = jnp.max(z, axis=-1, keepdims=True)
    zs = z - zmax
    out_ref[...] = zs - jnp.log(jnp.sum(jnp.exp(zs), axis=-1, keepdims=True))


def kernel(x, edge_index, W1, b1, W2, b2):
    n, nfeat = x.shape
    nhid = W1.shape[1]
    ncls = W2.shape[1]
    bn = 2000
    nblk = n // bn
    assert nblk * bn == n

    src = edge_index[0].astype(jnp.int32)
    dst = edge_index[1].astype(jnp.int32)
    e = src.shape[0]
    _, n_pad = _row_pad(n)

    iters = -(-e // (_NW * _CH * _NBUF)) * _NBUF
    e_pad = _NW * iters * _CH
    pk = jnp.arange(e_pad - e, dtype=jnp.int32)
    trash = n + pk % (n_pad - n)
    src_gat3 = jnp.concatenate([src, pk % n]).reshape(_NW, iters, _CH)
    src_cnt3 = jnp.concatenate([src, trash]).reshape(_NW, iters, _CH)
    dst3 = jnp.concatenate([dst, trash]).reshape(_NW, iters, _CH)

    odegs, idegs = _sc_bincount2(src_cnt3, dst3, n)

    odeg_spec = pl.BlockSpec((2, bn, 16), lambda i: (0, i, 0))
    ideg_spec = pl.BlockSpec((2, bn, 16), lambda i: (0, i, 0))

    m1 = pl.pallas_call(
        _tc_a_body,
        grid=(nblk,),
        in_specs=[
            pl.BlockSpec((bn, nfeat), lambda i: (i, 0)),
            pl.BlockSpec((nfeat, nhid), lambda i: (0, 0)),
            odeg_spec,
        ],
        out_specs=pl.BlockSpec((bn, nhid), lambda i: (i, 0)),
        out_shape=jax.ShapeDtypeStruct((n, nhid), jnp.float32),
    )(x, W1, odegs)

    part1 = _sc_scatter(m1, src_gat3, dst3)

    m2 = pl.pallas_call(
        _tc_b_body,
        grid=(nblk,),
        in_specs=[
            pl.BlockSpec((2, bn, nhid), lambda i: (0, i, 0)),
            odeg_spec,
            ideg_spec,
            pl.BlockSpec((1, nhid), lambda i: (0, 0)),
            pl.BlockSpec((nhid, ncls), lambda i: (0, 0)),
        ],
        out_specs=pl.BlockSpec((bn, ncls), lambda i: (i, 0)),
        out_shape=jax.ShapeDtypeStruct((n, ncls), jnp.float32),
    )(part1, odegs, idegs, b1.reshape(1, nhid), W2)

    part2 = _sc_scatter(m2, src_gat3, dst3)

    out = pl.pallas_call(
        _tc_c_body,
        grid=(nblk,),
        in_specs=[
            pl.BlockSpec((2, bn, ncls), lambda i: (0, i, 0)),
            ideg_spec,
            pl.BlockSpec((1, ncls), lambda i: (0, 0)),
        ],
        out_specs=pl.BlockSpec((bn, ncls), lambda i: (i, 0)),
        out_shape=jax.ShapeDtypeStruct((n, ncls), jnp.float32),
    )(part2, idegs, b2.reshape(1, ncls))

    return out


# speedup vs baseline: 1.2941x; 1.0001x over previous
import functools

import jax
import jax.numpy as jnp
from jax import lax
from jax.experimental import pallas as pl
from jax.experimental.pallas import tpu as pltpu
from jax.experimental.pallas import tpu_sc as plsc

_NC = 2
_NS = 16
_NW = _NC * _NS
_CH = 112
_NBUF = 2


def _row_pad(n):
    rows_per_tile = -(-n // (8 * _NS)) * 8
    return rows_per_tile, rows_per_tile * _NS


def _sc_bincount2(sidx3, didx3, n):
    nw, iters, ch = sidx3.shape
    assert nw == _NW and ch == _CH
    rows_per_tile, n_pad = _row_pad(n)

    mesh = plsc.VectorSubcoreMesh(core_axis_name="c", subcore_axis_name="s")

    @functools.partial(
        pl.kernel,
        mesh=mesh,
        compiler_params=pltpu.CompilerParams(use_tc_tiling_on_sc=False),
        out_type=[jax.ShapeDtypeStruct((2, n_pad, 16), jnp.float32),
                  jax.ShapeDtypeStruct((2, n_pad, 16), jnp.float32)],
        scratch_types=[
            pltpu.VMEM((iters, _CH), jnp.int32),
            pltpu.VMEM((iters, _CH), jnp.int32),
            pltpu.VMEM((_CH, 16), jnp.float32),
            pltpu.VMEM_SHARED((n_pad, 16), jnp.float32),
            pltpu.VMEM_SHARED((n_pad, 16), jnp.float32),
            pltpu.SemaphoreType.DMA,
            pltpu.SemaphoreType.DMA,
        ],
    )
    def k(sidx_hbm, didx_hbm, zeros_hbm, outs_hbm, outd_hbm, sidx, didx, ones,
          acc_s, acc_d, sem_s, sem_d):
        c = lax.axis_index("c")
        s = lax.axis_index("s")
        wid = c * _NS + s
        base_rows = s * rows_per_tile

        def fill_ones(i, carry):
            ones[i] = jnp.ones((16,), jnp.float32)
            return carry

        lax.fori_loop(0, _CH, fill_ones, 0)

        rt = pl.ds(base_rows, rows_per_tile)
        c1 = pltpu.async_copy(zeros_hbm, acc_s.at[rt], sem_s)
        c2 = pltpu.async_copy(zeros_hbm, acc_d.at[rt], sem_d)
        c3 = pltpu.async_copy(sidx_hbm.at[wid], sidx, sem_s)
        c4 = pltpu.async_copy(didx_hbm.at[wid], didx, sem_d)
        c1.wait(); c2.wait(); c3.wait(); c4.wait()
        plsc.subcore_barrier()

        def step(i, carry):
            pltpu.async_copy(ones, acc_s.at[sidx.at[i]], sem_s, add=True)
            pltpu.async_copy(ones, acc_d.at[didx.at[i]], sem_d, add=True)

            @pl.when(i > 0)
            def _():
                pltpu.make_async_copy(ones, acc_s.at[sidx.at[i]], sem_s).wait()
                pltpu.make_async_copy(ones, acc_d.at[didx.at[i]], sem_d).wait()

            return carry

        lax.fori_loop(0, iters, step, 0)
        pltpu.make_async_copy(ones, acc_s.at[sidx.at[0]], sem_s).wait()
        pltpu.make_async_copy(ones, acc_d.at[didx.at[0]], sem_d).wait()

        plsc.subcore_barrier()
        rt = pl.ds(base_rows, rows_per_tile)
        pltpu.sync_copy(acc_s.at[rt], outs_hbm.at[c, rt])
        pltpu.sync_copy(acc_d.at[rt], outd_hbm.at[c, rt])

    return k(sidx3, didx3, jnp.zeros((rows_per_tile, 16), jnp.float32))


def _sc_scatter(m, sidx3, didx3):
    n, f = m.shape
    nw, iters, ch = sidx3.shape
    rows_per_tile, n_pad = _row_pad(n)
    nbuf = _NBUF
    for cand in (4, 3):
        words = 16 * (2 * iters * _CH + cand * _CH * f) + n_pad * f
        if iters % cand == 0 and words <= 1_900_000:
            nbuf = cand
            break
    assert nw == _NW and ch == _CH and iters % nbuf == 0

    mesh = plsc.VectorSubcoreMesh(core_axis_name="c", subcore_axis_name="s")

    @functools.partial(
        pl.kernel,
        mesh=mesh,
        compiler_params=pltpu.CompilerParams(use_tc_tiling_on_sc=False),
        out_type=jax.ShapeDtypeStruct((2, n_pad, f), jnp.float32),
        scratch_types=[
            pltpu.VMEM((iters, _CH), jnp.int32),
            pltpu.VMEM((iters, _CH), jnp.int32),
        ] + [pltpu.VMEM((_CH, f), jnp.float32)] * nbuf + [
            pltpu.VMEM_SHARED((n_pad, f), jnp.float32),
        ] + [pltpu.SemaphoreType.DMA] * nbuf,
    )
    def k(m_hbm, src_hbm, dst_hbm, zeros_hbm, out_hbm, sidx, didx, *rest):
        rows = rest[:nbuf]
        acc = rest[nbuf]
        semg = rest[nbuf + 1:2 * nbuf + 1]
        c = lax.axis_index("c")
        s = lax.axis_index("s")
        wid = c * _NS + s
        base_rows = s * rows_per_tile

        c1 = pltpu.async_copy(zeros_hbm, acc.at[pl.ds(base_rows,
                                                      rows_per_tile)], semg[0])
        c2 = pltpu.async_copy(src_hbm.at[wid], sidx, semg[0])
        c3 = pltpu.async_copy(dst_hbm.at[wid], didx, semg[1])
        c1.wait(); c2.wait(); c3.wait()
        plsc.subcore_barrier()

        def gather_start(i, b):
            pltpu.async_copy(m_hbm.at[sidx.at[i]], rows[b], semg[b])

        def gather_wait(i, b):
            pltpu.make_async_copy(m_hbm.at[sidx.at[i]], rows[b], semg[b]).wait()

        for b in range(nbuf):
            gather_start(b, b)

        def grp(j, carry):
            i0 = j * nbuf
            for b in range(nbuf):
                gather_wait(i0 + b, b)
                pltpu.sync_copy(rows[b], acc.at[didx.at[i0 + b]], add=True)

                @pl.when(i0 + b + nbuf < iters)
                def _(b=b):
                    gather_start(i0 + b + nbuf, b)

            return carry

        lax.fori_loop(0, iters // nbuf, grp, 0)

        plsc.subcore_barrier()
        rt = pl.ds(base_rows, rows_per_tile)
        pltpu.sync_copy(acc.at[rt], out_hbm.at[c, rt])

    return k(m, sidx3, didx3, jnp.zeros((rows_per_tile, f), jnp.float32))


def _onorm(degs_ref):
    od = degs_ref[0, :, 0:1] + degs_ref[1, :, 0:1]
    return lax.rsqrt(jnp.maximum(od, 1.0))


def _tc_a_body(x_ref, w_ref, degs_ref, m1_ref):
    m1_ref[...] = jnp.dot(
        x_ref[...] * _onorm(degs_ref), w_ref[...],
        preferred_element_type=jnp.float32)


def _tc_b_body(p1_ref, odeg_ref, ideg_ref, b1_ref, w2_ref, m2_ref):
    inorm = _onorm(ideg_ref)
    h = (p1_ref[0] + p1_ref[1]) * inorm + b1_ref[...]
    h = jnp.maximum(h, 0.0)
    m2_ref[...] = jnp.dot(
        h, w2_ref[...], preferred_element_type=jnp.float32) * _onorm(odeg_ref)


def _tc_c_body(p2_ref, ideg_ref, b2_ref, out_ref):
    z = (p2_ref[0] + p2_ref[1]) * _onorm(ideg_ref) + b2_ref[...]
    zmax = jnp.max(z, axis=-1, keepdims=True)
    zs = z - zmax
    out_ref[...] = zs - jnp.log(jnp.sum(jnp.exp(zs), axis=-1, keepdims=True))


def kernel(x, edge_index, W1, b1, W2, b2):
    n, nfeat = x.shape
    nhid = W1.shape[1]
    ncls = W2.shape[1]
    bn = 2000
    nblk = n // bn
    assert nblk * bn == n

    src = edge_index[0].astype(jnp.int32)
    dst = edge_index[1].astype(jnp.int32)
    e = src.shape[0]
    _, n_pad = _row_pad(n)

    iters = -(-e // (_NW * _CH * _NBUF)) * _NBUF
    e_pad = _NW * iters * _CH
    pk = jnp.arange(e_pad - e, dtype=jnp.int32)
    trash = n + pk % (n_pad - n)
    src_gat3 = jnp.concatenate([src, pk % n]).reshape(_NW, iters, _CH)
    src_cnt3 = jnp.concatenate([src, trash]).reshape(_NW, iters, _CH)
    dst3 = jnp.concatenate([dst, trash]).reshape(_NW, iters, _CH)

    odegs, idegs = _sc_bincount2(src_cnt3, dst3, n)

    odeg_spec = pl.BlockSpec((2, bn, 16), lambda i: (0, i, 0))
    ideg_spec = pl.BlockSpec((2, bn, 16), lambda i: (0, i, 0))

    m1 = pl.pallas_call(
        _tc_a_body,
        grid=(nblk,),
        in_specs=[
            pl.BlockSpec((bn, nfeat), lambda i: (i, 0)),
            pl.BlockSpec((nfeat, nhid), lambda i: (0, 0)),
            odeg_spec,
        ],
        out_specs=pl.BlockSpec((bn, nhid), lambda i: (i, 0)),
        out_shape=jax.ShapeDtypeStruct((n, nhid), jnp.float32),
    )(x, W1, odegs)

    part1 = _sc_scatter(m1, src_gat3, dst3)

    m2 = pl.pallas_call(
        _tc_b_body,
        grid=(nblk,),
        in_specs=[
            pl.BlockSpec((2, bn, nhid), lambda i: (0, i, 0)),
            odeg_spec,
            ideg_spec,
            pl.BlockSpec((1, nhid), lambda i: (0, 0)),
            pl.BlockSpec((nhid, ncls), lambda i: (0, 0)),
        ],
        out_specs=pl.BlockSpec((bn, ncls), lambda i: (i, 0)),
        out_shape=jax.ShapeDtypeStruct((n, ncls), jnp.float32),
    )(part1, odegs, idegs, b1.reshape(1, nhid), W2)

    part2 = _sc_scatter(m2, src_gat3, dst3)

    out = pl.pallas_call(
        _tc_c_body,
        grid=(nblk,),
        in_specs=[
            pl.BlockSpec((2, bn, ncls), lambda i: (0, i, 0)),
            ideg_spec,
            pl.BlockSpec((1, ncls), lambda i: (0, 0)),
        ],
        out_specs=pl.BlockSpec((bn, ncls), lambda i: (i, 0)),
        out_shape=jax.ShapeDtypeStruct((n, ncls), jnp.float32),
    )(part2, idegs, b2.reshape(1, ncls))

    return out


# speedup vs baseline: 1.3000x; 1.0045x over previous
import functools

import jax
import jax.numpy as jnp
from jax import lax
from jax.experimental import pallas as pl
from jax.experimental.pallas import tpu as pltpu
from jax.experimental.pallas import tpu_sc as plsc

_NC = 2
_NS = 16
_NW = _NC * _NS
_CH = 112
_NBUF = 2


def _row_pad(n):
    rows_per_tile = -(-n // (8 * _NS)) * 8
    return rows_per_tile, rows_per_tile * _NS


def _sc_bincount2(sidx3, didx3, n):
    nw, iters, ch = sidx3.shape
    assert nw == _NW and ch == _CH
    rows_per_tile, n_pad = _row_pad(n)

    mesh = plsc.VectorSubcoreMesh(core_axis_name="c", subcore_axis_name="s")

    @functools.partial(
        pl.kernel,
        mesh=mesh,
        compiler_params=pltpu.CompilerParams(use_tc_tiling_on_sc=False),
        out_type=[jax.ShapeDtypeStruct((2, n_pad, 16), jnp.float32),
                  jax.ShapeDtypeStruct((2, n_pad, 16), jnp.float32)],
        scratch_types=[
            pltpu.VMEM((iters, _CH), jnp.int32),
            pltpu.VMEM((iters, _CH), jnp.int32),
            pltpu.VMEM((_CH, 16), jnp.float32),
            pltpu.VMEM_SHARED((n_pad, 16), jnp.float32),
            pltpu.VMEM_SHARED((n_pad, 16), jnp.float32),
            pltpu.SemaphoreType.DMA,
            pltpu.SemaphoreType.DMA,
        ],
    )
    def k(sidx_hbm, didx_hbm, zeros_hbm, outs_hbm, outd_hbm, sidx, didx, ones,
          acc_s, acc_d, sem_s, sem_d):
        c = lax.axis_index("c")
        s = lax.axis_index("s")
        wid = c * _NS + s
        base_rows = s * rows_per_tile

        def fill_ones(i, carry):
            ones[i] = jnp.ones((16,), jnp.float32)
            return carry

        lax.fori_loop(0, _CH, fill_ones, 0)

        rt = pl.ds(base_rows, rows_per_tile)
        c1 = pltpu.async_copy(zeros_hbm, acc_s.at[rt], sem_s)
        c2 = pltpu.async_copy(zeros_hbm, acc_d.at[rt], sem_d)
        c3 = pltpu.async_copy(sidx_hbm.at[wid], sidx, sem_s)
        c4 = pltpu.async_copy(didx_hbm.at[wid], didx, sem_d)
        c1.wait(); c2.wait(); c3.wait(); c4.wait()
        plsc.subcore_barrier()

        def step(i, carry):
            pltpu.async_copy(ones, acc_s.at[sidx.at[i]], sem_s, add=True)
            pltpu.async_copy(ones, acc_d.at[didx.at[i]], sem_d, add=True)

            @pl.when(i > 0)
            def _():
                pltpu.make_async_copy(ones, acc_s.at[sidx.at[i]], sem_s).wait()
                pltpu.make_async_copy(ones, acc_d.at[didx.at[i]], sem_d).wait()

            return carry

        lax.fori_loop(0, iters, step, 0)
        pltpu.make_async_copy(ones, acc_s.at[sidx.at[0]], sem_s).wait()
        pltpu.make_async_copy(ones, acc_d.at[didx.at[0]], sem_d).wait()

        plsc.subcore_barrier()
        rt = pl.ds(base_rows, rows_per_tile)
        pltpu.sync_copy(acc_s.at[rt], outs_hbm.at[c, rt])
        pltpu.sync_copy(acc_d.at[rt], outd_hbm.at[c, rt])

    return k(sidx3, didx3, jnp.zeros((rows_per_tile, 16), jnp.float32))


def _sc_scatter(m, sidx3, didx3):
    n, f = m.shape
    nw, iters, ch = sidx3.shape
    rows_per_tile, n_pad = _row_pad(n)
    nbuf = _NBUF
    for cand in (4, 3):
        words = 16 * (2 * iters * _CH + cand * _CH * f) + n_pad * f
        if iters % cand == 0 and words <= 1_900_000:
            nbuf = cand
            break
    assert nw == _NW and ch == _CH and iters % nbuf == 0

    mesh = plsc.VectorSubcoreMesh(core_axis_name="c", subcore_axis_name="s")

    @functools.partial(
        pl.kernel,
        mesh=mesh,
        compiler_params=pltpu.CompilerParams(use_tc_tiling_on_sc=False),
        out_type=jax.ShapeDtypeStruct((2, n_pad, f), jnp.float32),
        scratch_types=[
            pltpu.VMEM((iters, _CH), jnp.int32),
            pltpu.VMEM((iters, _CH), jnp.int32),
        ] + [pltpu.VMEM((_CH, f), jnp.float32)] * nbuf + [
            pltpu.VMEM_SHARED((n_pad, f), jnp.float32),
        ] + [pltpu.SemaphoreType.DMA] * nbuf,
    )
    def k(m_hbm, src_hbm, dst_hbm, zeros_hbm, out_hbm, sidx, didx, *rest):
        rows = rest[:nbuf]
        acc = rest[nbuf]
        semg = rest[nbuf + 1:2 * nbuf + 1]
        c = lax.axis_index("c")
        s = lax.axis_index("s")
        wid = c * _NS + s
        base_rows = s * rows_per_tile

        c1 = pltpu.async_copy(zeros_hbm, acc.at[pl.ds(base_rows,
                                                      rows_per_tile)], semg[0])
        c2 = pltpu.async_copy(src_hbm.at[wid], sidx, semg[0])
        c3 = pltpu.async_copy(dst_hbm.at[wid], didx, semg[1])
        c1.wait(); c2.wait(); c3.wait()
        plsc.subcore_barrier()

        def gather_start(i, b):
            pltpu.async_copy(m_hbm.at[sidx.at[i]], rows[b], semg[b])

        def gather_wait(i, b):
            pltpu.make_async_copy(m_hbm.at[sidx.at[i]], rows[b], semg[b]).wait()

        for b in range(nbuf):
            gather_start(b, b)

        def grp(j, carry):
            i0 = j * nbuf
            for b in range(nbuf):
                gather_wait(i0 + b, b)
                pltpu.sync_copy(rows[b], acc.at[didx.at[i0 + b]], add=True)

                @pl.when(i0 + b + nbuf < iters)
                def _(b=b):
                    gather_start(i0 + b + nbuf, b)

            return carry

        lax.fori_loop(0, iters // nbuf, grp, 0)

        plsc.subcore_barrier()
        rt = pl.ds(base_rows, rows_per_tile)
        pltpu.sync_copy(acc.at[rt], out_hbm.at[c, rt])

    return k(m, sidx3, didx3, jnp.zeros((rows_per_tile, f), jnp.float32))


def _onorm(degs_ref):
    od = degs_ref[0, :, 0:1] + degs_ref[1, :, 0:1]
    return lax.rsqrt(jnp.maximum(od, 1.0))


def _tc_a_body(x_ref, w_ref, degs_ref, m1_ref):
    m1_ref[...] = jnp.dot(
        x_ref[...] * _onorm(degs_ref), w_ref[...],
        preferred_element_type=jnp.float32)


def _tc_b_body(p1_ref, odeg_ref, ideg_ref, b1_ref, w2_ref, m2_ref):
    inorm = _onorm(ideg_ref)
    h = (p1_ref[0] + p1_ref[1]) * inorm + b1_ref[...]
    h = jnp.maximum(h, 0.0)
    m2_ref[...] = jnp.dot(
        h, w2_ref[...], preferred_element_type=jnp.float32) * _onorm(odeg_ref)


def _tc_c_body(p2_ref, ideg_ref, b2_ref, out_ref):
    z = (p2_ref[0] + p2_ref[1]) * _onorm(ideg_ref) + b2_ref[...]
    zmax = jnp.max(z, axis=-1, keepdims=True)
    zs = z - zmax
    out_ref[...] = zs - jnp.log(jnp.sum(jnp.exp(zs), axis=-1, keepdims=True))


def kernel(x, edge_index, W1, b1, W2, b2):
    n, nfeat = x.shape
    nhid = W1.shape[1]
    ncls = W2.shape[1]
    bn = 2000
    nblk = n // bn
    assert nblk * bn == n

    src = edge_index[0].astype(jnp.int32)
    dst = edge_index[1].astype(jnp.int32)
    e = src.shape[0]
    _, n_pad = _row_pad(n)

    iters = -(-e // (_NW * _CH * 4)) * 4
    e_pad = _NW * iters * _CH
    pk = jnp.arange(e_pad - e, dtype=jnp.int32)
    trash = n + pk % (n_pad - n)
    src_gat3 = jnp.concatenate([src, pk % n]).reshape(_NW, iters, _CH)
    src_cnt3 = jnp.concatenate([src, trash]).reshape(_NW, iters, _CH)
    dst3 = jnp.concatenate([dst, trash]).reshape(_NW, iters, _CH)

    odegs, idegs = _sc_bincount2(src_cnt3, dst3, n)

    odeg_spec = pl.BlockSpec((2, bn, 16), lambda i: (0, i, 0))
    ideg_spec = pl.BlockSpec((2, bn, 16), lambda i: (0, i, 0))

    m1 = pl.pallas_call(
        _tc_a_body,
        grid=(nblk,),
        in_specs=[
            pl.BlockSpec((bn, nfeat), lambda i: (i, 0)),
            pl.BlockSpec((nfeat, nhid), lambda i: (0, 0)),
            odeg_spec,
        ],
        out_specs=pl.BlockSpec((bn, nhid), lambda i: (i, 0)),
        out_shape=jax.ShapeDtypeStruct((n, nhid), jnp.float32),
    )(x, W1, odegs)

    part1 = _sc_scatter(m1, src_gat3, dst3)

    m2 = pl.pallas_call(
        _tc_b_body,
        grid=(nblk,),
        in_specs=[
            pl.BlockSpec((2, bn, nhid), lambda i: (0, i, 0)),
            odeg_spec,
            ideg_spec,
            pl.BlockSpec((1, nhid), lambda i: (0, 0)),
            pl.BlockSpec((nhid, ncls), lambda i: (0, 0)),
        ],
        out_specs=pl.BlockSpec((bn, ncls), lambda i: (i, 0)),
        out_shape=jax.ShapeDtypeStruct((n, ncls), jnp.float32),
    )(part1, odegs, idegs, b1.reshape(1, nhid), W2)

    part2 = _sc_scatter(m2, src_gat3, dst3)

    out = pl.pallas_call(
        _tc_c_body,
        grid=(nblk,),
        in_specs=[
            pl.BlockSpec((2, bn, ncls), lambda i: (0, i, 0)),
            ideg_spec,
            pl.BlockSpec((1, ncls), lambda i: (0, 0)),
        ],
        out_specs=pl.BlockSpec((bn, ncls), lambda i: (i, 0)),
        out_shape=jax.ShapeDtypeStruct((n, ncls), jnp.float32),
    )(part2, idegs, b2.reshape(1, ncls))

    return out
